# initial kernel scaffold (unmeasured)
import jax
import jax.numpy as jnp
from jax import lax
from jax.experimental import pallas as pl
from jax.experimental.pallas import tpu as pltpu

N_X = 2
E_LOCAL = 4
CAP = 640


def _allgather_x(x_bf, assign2d):
    m, d = x_bf.shape
    ar, ac = assign2d.shape

    def body(x_ref, a_ref, xall_ref, aall_ref, send_sems, recv_sems):
        my_x = lax.axis_index("x")
        my_y = lax.axis_index("y")
        peer = (1 - my_x, my_y)

        barrier = pltpu.get_barrier_semaphore()
        pl.semaphore_signal(
            barrier, inc=1, device_id=peer, device_id_type=pl.DeviceIdType.MESH
        )
        pl.semaphore_wait(barrier, 1)

        xall_ref[pl.ds(my_x * m, m), :] = x_ref[...]
        aall_ref[pl.ds(my_x * ar, ar), :] = a_ref[...]

        rdma_x = pltpu.make_async_remote_copy(
            src_ref=x_ref,
            dst_ref=xall_ref.at[pl.ds(my_x * m, m), :],
            send_sem=send_sems.at[0],
            recv_sem=recv_sems.at[0],
            device_id=peer,
            device_id_type=pl.DeviceIdType.MESH,
        )
        rdma_a = pltpu.make_async_remote_copy(
            src_ref=a_ref,
            dst_ref=aall_ref.at[pl.ds(my_x * ar, ar), :],
            send_sem=send_sems.at[1],
            recv_sem=recv_sems.at[1],
            device_id=peer,
            device_id_type=pl.DeviceIdType.MESH,
        )
        rdma_x.start()
        rdma_a.start()
        rdma_x.wait()
        rdma_a.wait()

    return pl.pallas_call(
        body,
        out_shape=[
            jax.ShapeDtypeStruct((N_X * m, d), x_bf.dtype),
            jax.ShapeDtypeStruct((N_X * ar, ac), assign2d.dtype),
        ],
        in_specs=[
            pl.BlockSpec(memory_space=pltpu.VMEM),
            pl.BlockSpec(memory_space=pltpu.VMEM),
        ],
        out_specs=[
            pl.BlockSpec(memory_space=pltpu.VMEM),
            pl.BlockSpec(memory_space=pltpu.VMEM),
        ],
        scratch_shapes=[
            pltpu.SemaphoreType.DMA((2,)),
            pltpu.SemaphoreType.DMA((2,)),
        ],
        compiler_params=pltpu.CompilerParams(collective_id=0),
    )(x_bf, assign2d)


def _expert_ffn(xe, w1, w2):
    e, cap, d = xe.shape
    f = w1.shape[2]

    def body(x_ref, w1_ref, w2_ref, o_ref):
        h = jnp.maximum(
            jnp.dot(x_ref[0], w1_ref[0], preferred_element_type=jnp.float32),
            0.0,
        ).astype(jnp.bfloat16)
        o_ref[0] = jnp.dot(
            h, w2_ref[0], preferred_element_type=jnp.float32
        ).astype(jnp.bfloat16)

    return pl.pallas_call(
        body,
        grid=(e,),
        out_shape=jax.ShapeDtypeStruct((e, cap, d), jnp.bfloat16),
        in_specs=[
            pl.BlockSpec((1, cap, d), lambda j: (j, 0, 0)),
            pl.BlockSpec((1, d, f), lambda j: (j, 0, 0)),
            pl.BlockSpec((1, f, d), lambda j: (j, 0, 0)),
        ],
        out_specs=pl.BlockSpec((1, cap, d), lambda j: (j, 0, 0)),
    )(xe, w1, w2)


def _reduce_scatter(partial):
    mt, d = partial.shape
    m = mt // N_X

    def body(p_ref, out_ref, recv_buf, send_sem, recv_sem):
        my_x = lax.axis_index("x")
        my_y = lax.axis_index("y")
        peer = (1 - my_x, my_y)

        barrier = pltpu.get_barrier_semaphore()
        pl.semaphore_signal(
            barrier, inc=1, device_id=peer, device_id_type=pl.DeviceIdType.MESH
        )
        pl.semaphore_wait(barrier, 1)

        other = 1 - my_x
        rdma = pltpu.make_async_remote_copy(
            src_ref=p_ref.at[pl.ds(other * m, m), :],
            dst_ref=recv_buf,
            send_sem=send_sem,
            recv_sem=recv_sem,
            device_id=peer,
            device_id_type=pl.DeviceIdType.MESH,
        )
        rdma.start()
        out_ref[...] = p_ref[pl.ds(my_x * m, m), :].astype(jnp.float32)
        rdma.wait()
        out_ref[...] += recv_buf[...].astype(jnp.float32)

    return pl.pallas_call(
        body,
        out_shape=jax.ShapeDtypeStruct((m, d), jnp.float32),
        in_specs=[pl.BlockSpec(memory_space=pltpu.VMEM)],
        out_specs=pl.BlockSpec(memory_space=pltpu.VMEM),
        scratch_shapes=[
            pltpu.VMEM((m, d), partial.dtype),
            pltpu.SemaphoreType.DMA,
            pltpu.SemaphoreType.DMA,
        ],
        compiler_params=pltpu.CompilerParams(collective_id=1),
    )(partial)


def kernel(x, assign, W1, W2):
    m, d = x.shape
    n_tok = N_X * m

    x_bf = x.astype(jnp.bfloat16)
    assign2d = assign.reshape(8, m // 8)

    x_all, assign_all2d = _allgather_x(x_bf, assign2d)
    assign_all = assign_all2d.reshape(n_tok)

    my_x = lax.axis_index("x")
    e0 = my_x * E_LOCAL
    order = jnp.argsort(assign_all).astype(jnp.int32)
    counts = jnp.bincount(assign_all, length=8).astype(jnp.int32)
    starts = (jnp.cumsum(counts) - counts).astype(jnp.int32)
    order_pad = jnp.concatenate([order, jnp.zeros((CAP,), jnp.int32)])

    idx = jnp.stack(
        [
            lax.dynamic_slice(order_pad, (starts[e0 + j],), (CAP,))
            for j in range(E_LOCAL)
        ]
    )
    my_counts = lax.dynamic_slice(counts, (e0,), (E_LOCAL,))
    mask = jnp.arange(CAP)[None, :] < my_counts[:, None]

    xe = x_all[idx]

    ye = _expert_ffn(
        xe, W1.astype(jnp.bfloat16), W2.astype(jnp.bfloat16)
    )
    ye = jnp.where(mask[:, :, None], ye, jnp.bfloat16(0))

    partial = (
        jnp.zeros((n_tok, d), jnp.bfloat16)
        .at[idx.reshape(-1)]
        .add(ye.reshape(-1, d))
    )

    return _reduce_scatter(partial)


# baseline (device time: 195645 ns/iter reference)
import jax
import jax.numpy as jnp
from jax import lax
from jax.experimental import pallas as pl
from jax.experimental.pallas import tpu as pltpu

N_X = 2
E_LOCAL = 4
CAP = 640


def _allgather_x(x, assign2d):
    m, d = x.shape
    ar, ac = assign2d.shape

    def body(x_ref, a_ref, xall_ref, aall_ref, xbf, send_sems, recv_sems):
        my_x = lax.axis_index("x")
        my_y = lax.axis_index("y")
        peer = (1 - my_x, my_y)

        xbf[...] = x_ref[...].astype(jnp.bfloat16)

        barrier = pltpu.get_barrier_semaphore()
        pl.semaphore_signal(
            barrier, inc=1, device_id=peer, device_id_type=pl.DeviceIdType.MESH
        )
        pl.semaphore_wait(barrier, 1)

        rdma_x = pltpu.make_async_remote_copy(
            src_ref=xbf,
            dst_ref=xall_ref.at[pl.ds(my_x * m, m), :],
            send_sem=send_sems.at[0],
            recv_sem=recv_sems.at[0],
            device_id=peer,
            device_id_type=pl.DeviceIdType.MESH,
        )
        rdma_a = pltpu.make_async_remote_copy(
            src_ref=a_ref,
            dst_ref=aall_ref.at[pl.ds(my_x * ar, ar), :],
            send_sem=send_sems.at[1],
            recv_sem=recv_sems.at[1],
            device_id=peer,
            device_id_type=pl.DeviceIdType.MESH,
        )
        rdma_x.start()
        rdma_a.start()

        xall_ref[pl.ds(my_x * m, m), :] = xbf[...]
        aall_ref[pl.ds(my_x * ar, ar), :] = a_ref[...]

        rdma_x.wait()
        rdma_a.wait()

    return pl.pallas_call(
        body,
        out_shape=[
            jax.ShapeDtypeStruct((N_X * m, d), jnp.bfloat16),
            jax.ShapeDtypeStruct((N_X * ar, ac), assign2d.dtype),
        ],
        in_specs=[
            pl.BlockSpec(memory_space=pltpu.VMEM),
            pl.BlockSpec(memory_space=pltpu.VMEM),
        ],
        out_specs=[
            pl.BlockSpec(memory_space=pltpu.VMEM),
            pl.BlockSpec(memory_space=pltpu.VMEM),
        ],
        scratch_shapes=[
            pltpu.VMEM((m, d), jnp.bfloat16),
            pltpu.SemaphoreType.DMA((2,)),
            pltpu.SemaphoreType.DMA((2,)),
        ],
        compiler_params=pltpu.CompilerParams(collective_id=0),
    )(x, assign2d)


def _expert_ffn(xe, w1, w2):
    e, cap, d = xe.shape
    f = w1.shape[2]
    ft = 512
    n_ft = f // ft

    def body(x_ref, w1_ref, w2_ref, o_ref, acc):
        t = pl.program_id(1)
        w1b = w1_ref[0].astype(jnp.bfloat16)
        h = jnp.maximum(
            jnp.dot(x_ref[0], w1b, preferred_element_type=jnp.float32),
            0.0,
        ).astype(jnp.bfloat16)
        w2b = w2_ref[0].astype(jnp.bfloat16)
        p = jnp.dot(h, w2b, preferred_element_type=jnp.float32)

        @pl.when(t == 0)
        def _():
            acc[...] = p

        @pl.when(t > 0)
        def _():
            acc[...] += p

        @pl.when(t == n_ft - 1)
        def _():
            o_ref[0] = acc[...].astype(jnp.bfloat16)

    return pl.pallas_call(
        body,
        grid=(e, n_ft),
        out_shape=jax.ShapeDtypeStruct((e, cap, d), jnp.bfloat16),
        in_specs=[
            pl.BlockSpec((1, cap, d), lambda j, t: (j, 0, 0)),
            pl.BlockSpec((1, d, ft), lambda j, t: (j, 0, t)),
            pl.BlockSpec((1, ft, d), lambda j, t: (j, t, 0)),
        ],
        out_specs=pl.BlockSpec((1, cap, d), lambda j, t: (j, 0, 0)),
        scratch_shapes=[pltpu.VMEM((cap, d), jnp.float32)],
    )(xe, w1, w2)


def _combine(mine, other):
    m, d = mine.shape

    def body(mine_ref, other_ref, out_ref, recv_buf, send_sem, recv_sem):
        my_x = lax.axis_index("x")
        my_y = lax.axis_index("y")
        peer = (1 - my_x, my_y)

        barrier = pltpu.get_barrier_semaphore()
        pl.semaphore_signal(
            barrier, inc=1, device_id=peer, device_id_type=pl.DeviceIdType.MESH
        )
        pl.semaphore_wait(barrier, 1)

        rdma = pltpu.make_async_remote_copy(
            src_ref=other_ref,
            dst_ref=recv_buf,
            send_sem=send_sem,
            recv_sem=recv_sem,
            device_id=peer,
            device_id_type=pl.DeviceIdType.MESH,
        )
        rdma.start()
        out_ref[...] = mine_ref[...].astype(jnp.float32)
        rdma.wait()
        out_ref[...] += recv_buf[...].astype(jnp.float32)

    return pl.pallas_call(
        body,
        out_shape=jax.ShapeDtypeStruct((m, d), jnp.float32),
        in_specs=[
            pl.BlockSpec(memory_space=pltpu.VMEM),
            pl.BlockSpec(memory_space=pltpu.VMEM),
        ],
        out_specs=pl.BlockSpec(memory_space=pltpu.VMEM),
        scratch_shapes=[
            pltpu.VMEM((m, d), jnp.bfloat16),
            pltpu.SemaphoreType.DMA,
            pltpu.SemaphoreType.DMA,
        ],
        compiler_params=pltpu.CompilerParams(collective_id=1),
    )(mine, other)


def kernel(x, assign, W1, W2):
    m, d = x.shape
    n_tok = N_X * m

    assign2d = assign.reshape(8, m // 8)

    x_all, assign_all2d = _allgather_x(x, assign2d)
    assign_all = assign_all2d.reshape(n_tok)

    my_x = lax.axis_index("x")
    e0 = my_x * E_LOCAL
    order = jnp.argsort(assign_all).astype(jnp.int32)
    counts = jnp.bincount(assign_all, length=8).astype(jnp.int32)
    starts = (jnp.cumsum(counts) - counts).astype(jnp.int32)
    order_pad = jnp.concatenate([order, jnp.zeros((CAP,), jnp.int32)])

    idx = jnp.stack(
        [
            lax.dynamic_slice(order_pad, (starts[e0 + j],), (CAP,))
            for j in range(E_LOCAL)
        ]
    )
    my_counts = lax.dynamic_slice(counts, (e0,), (E_LOCAL,))
    mask = jnp.arange(CAP)[None, :] < my_counts[:, None]

    xe = x_all[idx]

    ye = _expert_ffn(xe, W1, W2)

    ye_pad = jnp.concatenate(
        [ye.reshape(E_LOCAL * CAP, d), jnp.zeros((1, d), jnp.bfloat16)]
    )
    tgt = jnp.where(mask.reshape(-1), idx.reshape(-1), n_tok)
    pos = (
        jnp.full((n_tok + 1,), E_LOCAL * CAP, jnp.int32)
        .at[tgt]
        .set(jnp.arange(E_LOCAL * CAP, dtype=jnp.int32))
    )[:n_tok].reshape(N_X, m)

    mine = ye_pad[lax.dynamic_index_in_dim(pos, my_x, 0, keepdims=False)]
    other = ye_pad[lax.dynamic_index_in_dim(pos, 1 - my_x, 0, keepdims=False)]

    return _combine(mine, other)
